# baseline (speedup 1.0000x reference)
import numpy as np
import jax
import jax.numpy as jnp
from jax.experimental import pallas as pl
from jax.experimental.pallas import tpu as pltpu

S = 128
K = 64
RB = 16

_TINY = np.float32(np.finfo(np.float32).tiny)
_ROT_A = (13, 15, 26, 6)
_ROT_B = (17, 29, 16, 24)


def _np_threefry2x32(k0, k1, x0, x1):
    k0 = np.uint32(k0); k1 = np.uint32(k1)
    ks = (k0, k1, np.uint32(k0 ^ k1 ^ np.uint32(0x1BD11BDA)))
    x0 = np.uint32(np.uint64(x0) + ks[0]); x1 = np.uint32(np.uint64(x1) + ks[1])
    rots = (_ROT_A, _ROT_B, _ROT_A, _ROT_B, _ROT_A)
    inj = ((1, 2), (2, 0), (0, 1), (1, 2), (2, 0))
    for i in range(5):
        for r in rots[i]:
            x0 = np.uint32((np.uint64(x0) + np.uint64(x1)) & np.uint64(0xFFFFFFFF))
            x1 = np.uint32(((np.uint64(x1) << np.uint64(r)) | (np.uint64(x1) >> np.uint64(32 - r))) & np.uint64(0xFFFFFFFF))
            x1 = np.uint32(x1 ^ x0)
        a, b = inj[i]
        x0 = np.uint32((np.uint64(x0) + np.uint64(ks[a])) & np.uint64(0xFFFFFFFF))
        x1 = np.uint32((np.uint64(x1) + np.uint64(ks[b]) + np.uint64(i + 1)) & np.uint64(0xFFFFFFFF))
    return x0, x1


_KEY0, _KEY1 = _np_threefry2x32(0, 1, 0, 0)
_KEY2 = np.uint32(_KEY0 ^ _KEY1 ^ np.uint32(0x1BD11BDA))


def _threefry_bits(flat):
    ks = (np.uint32(_KEY0), np.uint32(_KEY1), _KEY2)
    rots = (_ROT_A, _ROT_B, _ROT_A, _ROT_B, _ROT_A)
    inj = ((1, 2), (2, 0), (0, 1), (1, 2), (2, 0))
    x0 = ks[0]
    x1 = flat + ks[1]
    for i in range(5):
        for r in rots[i]:
            x0 = x0 + x1
            x1 = (x1 << np.uint32(r)) | (x1 >> np.uint32(32 - r))
            x1 = x1 ^ x0
        a, b = inj[i]
        x0 = x0 + ks[a]
        x1 = x1 + np.uint32(np.uint64(ks[b]) + np.uint64(i + 1) & np.uint64(0xFFFFFFFF))
    return x0 ^ x1


def _resample_kernel(w_ref, tdl_ref, tdr_ref, sdl_ref, sdr_ref,
                     px_ref, py_ref, pz_ref,
                     inds_ref, tdlg_ref, tdrg_ref, sdlg_ref, sdrg_ref,
                     pxg_ref, pyg_ref, pzg_ref):
    blk = pl.program_id(0)
    base = blk * (RB * S * K)
    ri = jax.lax.broadcasted_iota(jnp.int32, (RB, K, S), 0)
    kk = jax.lax.broadcasted_iota(jnp.int32, (RB, K, S), 1)
    ss = jax.lax.broadcasted_iota(jnp.int32, (RB, K, S), 2)
    flat = (base + ri * (S * K) + ss * K + kk).astype(jnp.uint32)
    bits = _threefry_bits(flat)
    fl = jax.lax.bitcast_convert_type(
        (bits >> np.uint32(9)) | np.uint32(0x3F800000), jnp.float32) - np.float32(1.0)
    u = jnp.maximum(fl, _TINY)
    logw = jnp.log(jnp.maximum(w_ref[...], _TINY))
    vals = logw[:, None, :] - jnp.log(-jnp.log(u))
    m = jnp.max(vals, axis=-1, keepdims=True)
    ind = jnp.min(jnp.where(vals == m, ss, S), axis=-1)
    inds_ref[...] = ind
    oh = ss == ind[:, :, None]

    def gath(src_ref, out_ref):
        src = src_ref[...]
        out_ref[...] = jnp.sum(jnp.where(oh, src[:, None, :], np.float32(0.0)),
                               axis=-1)

    gath(tdl_ref, tdlg_ref)
    gath(tdr_ref, tdrg_ref)
    gath(sdl_ref, sdlg_ref)
    gath(sdr_ref, sdrg_ref)
    gath(px_ref, pxg_ref)
    gath(py_ref, pyg_ref)
    gath(pz_ref, pzg_ref)


def kernel(weights, points, tdist, sdist, num_resample):
    R = weights.shape[0]
    del num_resample
    tdl = jax.lax.slice_in_dim(tdist, 0, S, axis=1)
    tdr = jax.lax.slice_in_dim(tdist, 1, S + 1, axis=1)
    sdl = jax.lax.slice_in_dim(sdist, 0, S, axis=1)
    sdr = jax.lax.slice_in_dim(sdist, 1, S + 1, axis=1)
    px = points[:, :, 0]
    py = points[:, :, 1]
    pz = points[:, :, 2]

    in_spec = pl.BlockSpec((RB, S), lambda i: (i, 0))
    out_spec = pl.BlockSpec((RB, K), lambda i: (i, 0))
    outs = pl.pallas_call(
        _resample_kernel,
        grid=(R // RB,),
        in_specs=[in_spec] * 8,
        out_specs=[out_spec] * 8,
        out_shape=[jax.ShapeDtypeStruct((R, K), jnp.int32)] +
                  [jax.ShapeDtypeStruct((R, K), jnp.float32)] * 7,
        compiler_params=pltpu.CompilerParams(
            dimension_semantics=("arbitrary",)),
    )(weights, tdl, tdr, sdl, sdr, px, py, pz)
    inds, tdlg, tdrg, sdlg, sdrg, pxg, pyg, pzg = outs

    f_weights = jnp.ones((R, K), jnp.float32)
    f_points = jnp.stack([pxg, pyg, pzg], axis=-1)
    f_tdist = jnp.concatenate([tdlg, tdrg], axis=-1)
    f_sdist = jnp.concatenate([sdlg, sdrg], axis=-1)
    return (f_weights, f_points, f_tdist, f_sdist, inds)


# speedup vs baseline: 1.1428x; 1.1428x over previous
import numpy as np
import jax
import jax.numpy as jnp
from jax.experimental import pallas as pl
from jax.experimental.pallas import tpu as pltpu

S = 128
K = 64
RB = 32

_TINY = np.float32(np.finfo(np.float32).tiny)
_ROT_A = (13, 15, 26, 6)
_ROT_B = (17, 29, 16, 24)


def _np_threefry2x32(k0, k1, x0, x1):
    k0 = np.uint32(k0); k1 = np.uint32(k1)
    ks = (k0, k1, np.uint32(k0 ^ k1 ^ np.uint32(0x1BD11BDA)))
    x0 = np.uint32(np.uint64(x0) + ks[0]); x1 = np.uint32(np.uint64(x1) + ks[1])
    rots = (_ROT_A, _ROT_B, _ROT_A, _ROT_B, _ROT_A)
    inj = ((1, 2), (2, 0), (0, 1), (1, 2), (2, 0))
    for i in range(5):
        for r in rots[i]:
            x0 = np.uint32((np.uint64(x0) + np.uint64(x1)) & np.uint64(0xFFFFFFFF))
            x1 = np.uint32(((np.uint64(x1) << np.uint64(r)) | (np.uint64(x1) >> np.uint64(32 - r))) & np.uint64(0xFFFFFFFF))
            x1 = np.uint32(x1 ^ x0)
        a, b = inj[i]
        x0 = np.uint32((np.uint64(x0) + np.uint64(ks[a])) & np.uint64(0xFFFFFFFF))
        x1 = np.uint32((np.uint64(x1) + np.uint64(ks[b]) + np.uint64(i + 1)) & np.uint64(0xFFFFFFFF))
    return x0, x1


_KEY0, _KEY1 = _np_threefry2x32(0, 1, 0, 0)
_KEY2 = np.uint32(_KEY0 ^ _KEY1 ^ np.uint32(0x1BD11BDA))


def _threefry_bits(flat):
    ks = (np.uint32(_KEY0), np.uint32(_KEY1), _KEY2)
    rots = (_ROT_A, _ROT_B, _ROT_A, _ROT_B, _ROT_A)
    inj = ((1, 2), (2, 0), (0, 1), (1, 2), (2, 0))
    x0 = ks[0]
    x1 = flat + ks[1]
    for i in range(5):
        for r in rots[i]:
            x0 = x0 + x1
            x1 = (x1 << np.uint32(r)) | (x1 >> np.uint32(32 - r))
            x1 = x1 ^ x0
        a, b = inj[i]
        x0 = x0 + ks[a]
        x1 = x1 + np.uint32(np.uint64(ks[b]) + np.uint64(i + 1) & np.uint64(0xFFFFFFFF))
    return x0 ^ x1


def _resample_kernel(w_ref, src8_ref, inds_ref, out8_ref):
    blk = pl.program_id(0)
    base = blk * (RB * S * K)
    ri = jax.lax.broadcasted_iota(jnp.int32, (RB, K, S), 0)
    kk = jax.lax.broadcasted_iota(jnp.int32, (RB, K, S), 1)
    ss = jax.lax.broadcasted_iota(jnp.int32, (RB, K, S), 2)
    flat = (base + ri * (S * K) + ss * K + kk).astype(jnp.uint32)
    bits = _threefry_bits(flat)
    fl = jax.lax.bitcast_convert_type(
        (bits >> np.uint32(9)) | np.uint32(0x3F800000), jnp.float32) - np.float32(1.0)
    u = jnp.maximum(fl, _TINY)
    logw = jnp.log(jnp.maximum(w_ref[...], _TINY))
    vals = logw[:, None, :] - jnp.log(-jnp.log(u))
    m = jnp.max(vals, axis=-1, keepdims=True)
    ind = jnp.min(jnp.where(vals == m, ss, S), axis=-1)
    inds_ref[...] = ind
    ohf = jnp.where(ss == ind[:, :, None], np.float32(1.0), np.float32(0.0))
    for r in range(RB):
        out8_ref[r] = jax.lax.dot_general(
            src8_ref[r], ohf[r],
            dimension_numbers=(((1,), (1,)), ((), ())),
            precision=jax.lax.Precision.HIGHEST,
            preferred_element_type=jnp.float32)


def kernel(weights, points, tdist, sdist, num_resample):
    R = weights.shape[0]
    del num_resample
    tdl = jax.lax.slice_in_dim(tdist, 0, S, axis=1)
    tdr = jax.lax.slice_in_dim(tdist, 1, S + 1, axis=1)
    sdl = jax.lax.slice_in_dim(sdist, 0, S, axis=1)
    sdr = jax.lax.slice_in_dim(sdist, 1, S + 1, axis=1)
    px = points[:, :, 0]
    py = points[:, :, 1]
    pz = points[:, :, 2]
    zero = jnp.zeros_like(tdl)
    src8 = jnp.stack([tdl, tdr, sdl, sdr, px, py, pz, zero], axis=1)

    outs = pl.pallas_call(
        _resample_kernel,
        grid=(R // RB,),
        in_specs=[pl.BlockSpec((RB, S), lambda i: (i, 0)),
                  pl.BlockSpec((RB, 8, S), lambda i: (i, 0, 0))],
        out_specs=[pl.BlockSpec((RB, K), lambda i: (i, 0)),
                   pl.BlockSpec((RB, 8, K), lambda i: (i, 0, 0))],
        out_shape=[jax.ShapeDtypeStruct((R, K), jnp.int32),
                   jax.ShapeDtypeStruct((R, 8, K), jnp.float32)],
        compiler_params=pltpu.CompilerParams(
            dimension_semantics=("arbitrary",)),
    )(weights, src8)
    inds, out8 = outs

    f_weights = jnp.ones((R, K), jnp.float32)
    f_tdist = out8[:, 0:2, :].reshape(R, 2 * K)
    f_sdist = out8[:, 2:4, :].reshape(R, 2 * K)
    f_points = jnp.transpose(out8[:, 4:7, :], (0, 2, 1))
    return (f_weights, f_points, f_tdist, f_sdist, inds)


# speedup vs baseline: 1.1819x; 1.0342x over previous
import numpy as np
import jax
import jax.numpy as jnp
from jax.experimental import pallas as pl
from jax.experimental.pallas import tpu as pltpu

S = 128
K = 64
RB = 64

_TINY = np.float32(np.finfo(np.float32).tiny)
_ROT_A = (13, 15, 26, 6)
_ROT_B = (17, 29, 16, 24)


def _np_threefry2x32(k0, k1, x0, x1):
    k0 = np.uint32(k0); k1 = np.uint32(k1)
    ks = (k0, k1, np.uint32(k0 ^ k1 ^ np.uint32(0x1BD11BDA)))
    x0 = np.uint32(np.uint64(x0) + ks[0]); x1 = np.uint32(np.uint64(x1) + ks[1])
    rots = (_ROT_A, _ROT_B, _ROT_A, _ROT_B, _ROT_A)
    inj = ((1, 2), (2, 0), (0, 1), (1, 2), (2, 0))
    for i in range(5):
        for r in rots[i]:
            x0 = np.uint32((np.uint64(x0) + np.uint64(x1)) & np.uint64(0xFFFFFFFF))
            x1 = np.uint32(((np.uint64(x1) << np.uint64(r)) | (np.uint64(x1) >> np.uint64(32 - r))) & np.uint64(0xFFFFFFFF))
            x1 = np.uint32(x1 ^ x0)
        a, b = inj[i]
        x0 = np.uint32((np.uint64(x0) + np.uint64(ks[a])) & np.uint64(0xFFFFFFFF))
        x1 = np.uint32((np.uint64(x1) + np.uint64(ks[b]) + np.uint64(i + 1)) & np.uint64(0xFFFFFFFF))
    return x0, x1


_KEY0, _KEY1 = _np_threefry2x32(0, 1, 0, 0)
_KEY2 = np.uint32(_KEY0 ^ _KEY1 ^ np.uint32(0x1BD11BDA))


def _threefry_bits(flat):
    ks = (np.uint32(_KEY0), np.uint32(_KEY1), _KEY2)
    rots = (_ROT_A, _ROT_B, _ROT_A, _ROT_B, _ROT_A)
    inj = ((1, 2), (2, 0), (0, 1), (1, 2), (2, 0))
    x0 = ks[0]
    x1 = flat + ks[1]
    for i in range(5):
        for r in rots[i]:
            x0 = x0 + x1
            x1 = (x1 << np.uint32(r)) | (x1 >> np.uint32(32 - r))
            x1 = x1 ^ x0
        a, b = inj[i]
        x0 = x0 + ks[a]
        x1 = x1 + np.uint32(np.uint64(ks[b]) + np.uint64(i + 1) & np.uint64(0xFFFFFFFF))
    return x0 ^ x1


def _resample_kernel(w_ref, src8_ref, inds_ref, out8_ref):
    blk = pl.program_id(0)
    base = blk * (RB * S * K)
    ri = jax.lax.broadcasted_iota(jnp.int32, (RB, K, S), 0)
    kk = jax.lax.broadcasted_iota(jnp.int32, (RB, K, S), 1)
    ss = jax.lax.broadcasted_iota(jnp.int32, (RB, K, S), 2)
    flat = (base + ri * (S * K) + ss * K + kk).astype(jnp.uint32)
    bits = _threefry_bits(flat)
    fl = jax.lax.bitcast_convert_type(
        (bits >> np.uint32(9)) | np.uint32(0x3F800000), jnp.float32) - np.float32(1.0)
    u = jnp.maximum(fl, _TINY)
    logw = jnp.log(jnp.maximum(w_ref[...], _TINY))
    vals = logw[:, None, :] - jnp.log(-jnp.log(u))
    m = jnp.max(vals, axis=-1, keepdims=True)
    ind = jnp.min(jnp.where(vals == m, ss, S), axis=-1)
    inds_ref[...] = ind
    ohf = jnp.where(ss == ind[:, :, None], np.float32(1.0), np.float32(0.0))
    for r in range(RB):
        out8_ref[r] = jax.lax.dot_general(
            src8_ref[r], ohf[r],
            dimension_numbers=(((1,), (1,)), ((), ())),
            precision=jax.lax.Precision.HIGHEST,
            preferred_element_type=jnp.float32)


def kernel(weights, points, tdist, sdist, num_resample):
    R = weights.shape[0]
    del num_resample
    tdl = jax.lax.slice_in_dim(tdist, 0, S, axis=1)
    tdr = jax.lax.slice_in_dim(tdist, 1, S + 1, axis=1)
    sdl = jax.lax.slice_in_dim(sdist, 0, S, axis=1)
    sdr = jax.lax.slice_in_dim(sdist, 1, S + 1, axis=1)
    px = points[:, :, 0]
    py = points[:, :, 1]
    pz = points[:, :, 2]
    zero = jnp.zeros_like(tdl)
    src8 = jnp.stack([tdl, tdr, sdl, sdr, px, py, pz, zero], axis=1)

    outs = pl.pallas_call(
        _resample_kernel,
        grid=(R // RB,),
        in_specs=[pl.BlockSpec((RB, S), lambda i: (i, 0)),
                  pl.BlockSpec((RB, 8, S), lambda i: (i, 0, 0))],
        out_specs=[pl.BlockSpec((RB, K), lambda i: (i, 0)),
                   pl.BlockSpec((RB, 8, K), lambda i: (i, 0, 0))],
        out_shape=[jax.ShapeDtypeStruct((R, K), jnp.int32),
                   jax.ShapeDtypeStruct((R, 8, K), jnp.float32)],
        compiler_params=pltpu.CompilerParams(
            dimension_semantics=("parallel",)),
    )(weights, src8)
    inds, out8 = outs

    f_weights = jnp.ones((R, K), jnp.float32)
    f_tdist = out8[:, 0:2, :].reshape(R, 2 * K)
    f_sdist = out8[:, 2:4, :].reshape(R, 2 * K)
    f_points = jnp.transpose(out8[:, 4:7, :], (0, 2, 1))
    return (f_weights, f_points, f_tdist, f_sdist, inds)
